# baseline (speedup 1.0000x reference)
import functools

import jax
import jax.numpy as jnp
import numpy as np
from jax import lax
from jax.experimental import pallas as pl
from jax.experimental.pallas import tpu as pltpu
from jax.experimental.pallas import tpu_sc as plsc

VOCAB_N = 100000
HIDDEN_N = 768
MAX_POS_N = 4096
BATCH_N = 4
SEQ_N = 4096

_NC = 2
_NS = 16
_NW = _NC * _NS
_L = 16

_B = BATCH_N * SEQ_N
_PPW = SEQ_N // _NW
_CR = 32
_NJ = _PPW // _CR
_NK = _NJ * BATCH_N
_NV = HIDDEN_N // _L
_RU = 4


@functools.cache
def _scaledsin_table():
    pos = np.arange(MAX_POS_N, dtype=np.float32)
    half_d = HIDDEN_N // 2
    freq_seq = -np.arange(half_d, dtype=np.float32) / np.float32(half_d)
    inv_freq = (np.float32(10000.0) ** freq_seq).astype(np.float32)
    sinusoid = pos[:, None] * inv_freq[None, :]
    tab = np.concatenate([np.sin(sinusoid), np.cos(sinusoid)], axis=-1)
    return jax.device_put(tab.astype(np.float32))


_mesh = plsc.VectorSubcoreMesh(core_axis_name="c", subcore_axis_name="s")


@functools.partial(
    pl.kernel,
    out_type=jax.ShapeDtypeStruct((_B, HIDDEN_N), jnp.float32),
    mesh=_mesh,
    scratch_types=[
        pltpu.VMEM((BATCH_N * _PPW,), jnp.int32),
        pltpu.VMEM((_CR, HIDDEN_N), jnp.float32),
        pltpu.VMEM((_CR, HIDDEN_N), jnp.float32),
        pltpu.VMEM((_CR, HIDDEN_N), jnp.float32),
        pltpu.VMEM((_CR, HIDDEN_N), jnp.float32),
        pltpu.SemaphoreType.DMA,
        pltpu.SemaphoreType.DMA,
        pltpu.SemaphoreType.DMA,
        pltpu.SemaphoreType.DMA,
        pltpu.SemaphoreType.DMA,
        pltpu.SemaphoreType.DMA,
    ],
)
def _sc_embed(ids_hbm, tab_hbm, pos_hbm, out_hbm,
              idx_all, rows0, rows1, pos0, pos1,
              gsem0, gsem1, psem0, psem1, osem0, osem1):
    wid = lax.axis_index("s") * _NC + lax.axis_index("c")
    pbase = wid * _PPW

    rows = (rows0, rows1)
    gsems = (gsem0, gsem1)
    posb = (pos0, pos1)
    psems = (psem0, psem1)

    def start_pos(j):
        return pltpu.async_copy(pos_hbm.at[pl.ds(pbase + j * _CR, _CR)],
                                posb[j % 2], psems[j % 2])

    pos_pending = start_pos(0)

    for b in range(BATCH_N):
        pltpu.sync_copy(ids_hbm.at[pl.ds(b * SEQ_N + pbase, _PPW)],
                        idx_all.at[pl.ds(b * _PPW, _PPW)])

    def start_gather(kk):
        j, b = divmod(kk, BATCH_N)
        ioff = b * _PPW + j * _CR
        return pltpu.async_copy(tab_hbm.at[idx_all.at[pl.ds(ioff, _CR)]],
                                rows[kk % 2], gsems[kk % 2])

    pending = start_gather(0)

    osems = (osem0, osem1)
    out_pending = [None, None]

    for kk in range(_NK):
        j, b = divmod(kk, BATCH_N)
        cur = kk % 2
        buf = rows[cur]
        pv = posb[j % 2]
        if b == 0:
            pos_pending.wait()
        pending.wait()
        if kk + 1 < _NK:
            nxt = (kk + 1) % 2
            if out_pending[nxt] is not None:
                out_pending[nxt].wait()
                out_pending[nxt] = None
            pending = start_gather(kk + 1)
        if b == 0 and j + 1 < _NJ:
            pos_pending = start_pos(j + 1)

        def row_body(i, c2, buf=buf, pv=pv):
            r0 = i * _RU
            for u in range(_RU):
                for v in range(_NV):
                    sl = pl.ds(v * _L, _L)
                    plsc.addupdate(buf.at[r0 + u, sl], pv[r0 + u, sl])
            return c2

        lax.fori_loop(0, _CR // _RU, row_body, 0)
        cbase = b * SEQ_N + pbase + j * _CR
        out_pending[cur] = pltpu.async_copy(
            buf, out_hbm.at[pl.ds(cbase, _CR)], osems[cur])

    for h in out_pending:
        if h is not None:
            h.wait()


def kernel(input_ids, word_embeddings, scale):
    ids_flat = input_ids.reshape(-1).astype(jnp.int32)
    posemb = _scaledsin_table() * scale.astype(jnp.float32)[0]
    out = _sc_embed(ids_flat, word_embeddings, posemb)
    return out.reshape(BATCH_N, SEQ_N, HIDDEN_N)


# speedup vs baseline: 1.0502x; 1.0502x over previous
import functools

import jax
import jax.numpy as jnp
import numpy as np
from jax import lax
from jax.experimental import pallas as pl
from jax.experimental.pallas import tpu as pltpu
from jax.experimental.pallas import tpu_sc as plsc

VOCAB_N = 100000
HIDDEN_N = 768
MAX_POS_N = 4096
BATCH_N = 4
SEQ_N = 4096

_NC = 2
_NS = 16
_NW = _NC * _NS
_L = 16

_B = BATCH_N * SEQ_N
_PPW = SEQ_N // _NW
_CR = 32
_NJ = _PPW // _CR
_NK = _NJ * BATCH_N
_NV = HIDDEN_N // _L


@functools.cache
def _scaledsin_table():
    pos = np.arange(MAX_POS_N, dtype=np.float32)
    half_d = HIDDEN_N // 2
    freq_seq = -np.arange(half_d, dtype=np.float32) / np.float32(half_d)
    inv_freq = (np.float32(10000.0) ** freq_seq).astype(np.float32)
    sinusoid = pos[:, None] * inv_freq[None, :]
    tab = np.concatenate([np.sin(sinusoid), np.cos(sinusoid)], axis=-1)
    return jax.device_put(tab.astype(np.float32))


_mesh = plsc.VectorSubcoreMesh(core_axis_name="c", subcore_axis_name="s")


@functools.partial(
    pl.kernel,
    out_type=jax.ShapeDtypeStruct((_B, HIDDEN_N), jnp.float32),
    mesh=_mesh,
    scratch_types=[
        pltpu.VMEM((BATCH_N * _PPW,), jnp.int32),
        pltpu.VMEM((_CR, HIDDEN_N), jnp.float32),
        pltpu.VMEM((_CR, HIDDEN_N), jnp.float32),
        pltpu.VMEM((_CR, HIDDEN_N), jnp.float32),
        pltpu.VMEM((_CR, HIDDEN_N), jnp.float32),
        pltpu.VMEM((_CR, HIDDEN_N), jnp.float32),
        pltpu.SemaphoreType.DMA,
        pltpu.SemaphoreType.DMA,
        pltpu.SemaphoreType.DMA,
        pltpu.SemaphoreType.DMA,
        pltpu.SemaphoreType.DMA,
        pltpu.SemaphoreType.DMA,
        pltpu.SemaphoreType.DMA,
        pltpu.SemaphoreType.DMA,
    ],
)
def _sc_embed(ids_hbm, tab_hbm, pos_hbm, out_hbm,
              idx_all, rows0, rows1, rows2, pos0, pos1,
              gsem0, gsem1, gsem2, psem0, psem1, osem0, osem1, osem2):
    wid = lax.axis_index("s") * _NC + lax.axis_index("c")
    pbase = wid * _PPW

    rows = (rows0, rows1, rows2)
    gsems = (gsem0, gsem1, gsem2)
    posb = (pos0, pos1)
    psems = (psem0, psem1)

    def start_pos(j):
        return pltpu.async_copy(pos_hbm.at[pl.ds(pbase + j * _CR, _CR)],
                                posb[j % 2], psems[j % 2])

    pos_pending = start_pos(0)

    for b in range(BATCH_N):
        pltpu.sync_copy(ids_hbm.at[pl.ds(b * SEQ_N + pbase, _PPW)],
                        idx_all.at[pl.ds(b * _PPW, _PPW)])

    def start_gather(kk):
        j, b = divmod(kk, BATCH_N)
        ioff = b * _PPW + j * _CR
        return pltpu.async_copy(tab_hbm.at[idx_all.at[pl.ds(ioff, _CR)]],
                                rows[kk % 3], gsems[kk % 3])

    gat_pending = [start_gather(0), start_gather(1), None]

    osems = (osem0, osem1, osem2)
    out_pending = [None, None, None]

    for kk in range(_NK):
        j, b = divmod(kk, BATCH_N)
        cur = kk % 3
        buf = rows[cur]
        pv = posb[j % 2]
        if b == 0:
            pos_pending.wait()
        gat_pending[cur].wait()
        if kk + 2 < _NK:
            nx2 = (kk + 2) % 3
            if out_pending[nx2] is not None:
                out_pending[nx2].wait()
                out_pending[nx2] = None
            gat_pending[nx2] = start_gather(kk + 2)
        if b == 0 and j + 1 < _NJ:
            pos_pending = start_pos(j + 1)

        def row_body(r, c2, buf=buf, pv=pv):
            for v in range(_NV):
                sl = pl.ds(v * _L, _L)
                plsc.addupdate(buf.at[r, sl], pv[r, sl])
            return c2

        lax.fori_loop(0, _CR, row_body, 0)
        cbase = b * SEQ_N + pbase + j * _CR
        out_pending[cur] = pltpu.async_copy(
            buf, out_hbm.at[pl.ds(cbase, _CR)], osems[cur])

    for h in out_pending:
        if h is not None:
            h.wait()


def kernel(input_ids, word_embeddings, scale):
    ids_flat = input_ids.reshape(-1).astype(jnp.int32)
    posemb = _scaledsin_table() * scale.astype(jnp.float32)[0]
    out = _sc_embed(ids_flat, word_embeddings, posemb)
    return out.reshape(BATCH_N, SEQ_N, HIDDEN_N)
